# baseline (speedup 1.0000x reference)
import functools
import jax
import jax.numpy as jnp
from jax import lax
from jax.experimental import pallas as pl
from jax.experimental.pallas import tpu as pltpu
from jax.experimental.pallas import tpu_sc as plsc

E = 8
K = 2
H = 1024
I = 2048
T = 2048
TK = T * K

BM = 128
NB = 39
NPAD = NB * BM
MR = 40


NC = 2
NS = 16
NW = NC * NS
PW = TK // NW
CH = 32
NCH = PW // CH

@functools.lru_cache(maxsize=None)
def _make_dispatch():
    mesh = plsc.VectorSubcoreMesh(core_axis_name="c", subcore_axis_name="s")
    return functools.partial(
        pl.kernel,
        mesh=mesh,
        out_type=jax.ShapeDtypeStruct((NPAD, H), jnp.float32),
        scratch_types=[
            pltpu.VMEM((NCH, CH), jnp.int32),
            pltpu.VMEM((NCH, CH), jnp.int32),
            pltpu.VMEM((CH, H), jnp.float32),
            pltpu.VMEM((CH, H), jnp.float32),
            pltpu.SemaphoreType.DMA,
            pltpu.SemaphoreType.DMA,
            pltpu.SemaphoreType.DMA,
            pltpu.SemaphoreType.DMA,
        ],
    )(_dispatch_body)


def _dispatch_body(x_hbm, pos_hbm, tok_hbm, xs_hbm, pos_v, tok_v,
                   rows0, rows1, sg0, sg1, ss0, ss1):
    wid = lax.axis_index("s") * NC + lax.axis_index("c")
    bufs = (rows0, rows1)
    sgs = (sg0, sg1)
    sss = (ss0, ss1)
    pltpu.sync_copy(pos_hbm.at[wid], pos_v)
    pltpu.sync_copy(tok_hbm.at[wid], tok_v)
    gath = {0: pltpu.async_copy(x_hbm.at[tok_v.at[0]], bufs[0], sgs[0])}
    scat = {}
    for j in range(NCH):
        gath[j].wait()
        if j + 1 < NCH:
            if j - 1 >= 0:
                scat[j - 1].wait()
            gath[j + 1] = pltpu.async_copy(
                x_hbm.at[tok_v.at[j + 1]], bufs[(j + 1) % 2], sgs[(j + 1) % 2])
        scat[j] = pltpu.async_copy(
            bufs[j % 2], xs_hbm.at[pos_v.at[j]], sss[j % 2])
    scat[NCH - 2].wait()
    scat[NCH - 1].wait()


CT = CH // K
HV = H // 16


@functools.lru_cache(maxsize=None)
def _make_combine():
    mesh = plsc.VectorSubcoreMesh(core_axis_name="c", subcore_axis_name="s")
    return functools.partial(
        pl.kernel,
        mesh=mesh,
        out_type=jax.ShapeDtypeStruct((T, H), jnp.float32),
        scratch_types=[
            pltpu.VMEM((NCH, CH), jnp.int32),
            pltpu.VMEM((NCH, CH), jnp.float32),
            pltpu.VMEM((CH, H), jnp.float32),
            pltpu.VMEM((CH, H), jnp.float32),
            pltpu.VMEM((CT, H), jnp.float32),
            pltpu.SemaphoreType.DMA,
            pltpu.SemaphoreType.DMA,
        ],
    )(_combine_body)


def _combine_body(y_hbm, pos_hbm, w_hbm, out_hbm, pos_v, w_v,
                  rows0, rows1, out_v, sem0, sem1):
    wid = lax.axis_index("s") * NC + lax.axis_index("c")
    bufs = (rows0, rows1)
    sems = (sem0, sem1)
    pltpu.sync_copy(pos_hbm.at[wid], pos_v)
    pltpu.sync_copy(w_hbm.at[wid], w_v)
    gath = {0: pltpu.async_copy(y_hbm.at[pos_v.at[0]], bufs[0], sems[0])}
    for j in range(NCH):
        gath[j].wait()
        if j + 1 < NCH:
            gath[j + 1] = pltpu.async_copy(
                y_hbm.at[pos_v.at[j + 1]], bufs[(j + 1) % 2], sems[(j + 1) % 2])
        rows_v = bufs[j % 2]
        wrow_a = w_v[j, pl.ds(0, 16)]
        wrow_b = w_v[j, pl.ds(16, 16)]
        for tt in range(CT):
            wrow = wrow_a if tt < CT // 2 else wrow_b
            w0 = wrow[(2 * tt) % 16]
            w1 = wrow[(2 * tt + 1) % 16]

            def body(kk, _):
                a = rows_v[2 * tt, pl.ds(kk * 16, 16)]
                b = rows_v[2 * tt + 1, pl.ds(kk * 16, 16)]
                out_v[tt, pl.ds(kk * 16, 16)] = a * w0 + b * w1
                return 0

            lax.fori_loop(0, HV, body, 0)
        base = pl.multiple_of(wid * (PW // K) + j * CT, CT)
        pltpu.sync_copy(out_v, out_hbm.at[pl.ds(base, CT)])


def _routing_body(lg_ref, pos2_ref, w2o_ref, meta_ref):
    lg = lg_ref[...]
    m = jnp.max(lg, axis=1, keepdims=True)
    p = jnp.exp(lg - m)
    probs = p / jnp.sum(p, axis=1, keepdims=True)

    iota8 = lax.broadcasted_iota(jnp.int32, (T, E), 1)
    m1 = jnp.max(probs, axis=1, keepdims=True)
    i1 = jnp.min(jnp.where(probs == m1, iota8, E), axis=1, keepdims=True)
    probs2 = jnp.where(iota8 == i1, -1.0, probs)
    m2 = jnp.max(probs2, axis=1, keepdims=True)
    i2 = jnp.min(jnp.where(probs2 == m2, iota8, E), axis=1, keepdims=True)
    denom = m1 + m2
    w2o_ref[...] = jnp.concatenate([m1 / denom, m2 / denom], axis=1)

    cnt = ((iota8 == i1) | (iota8 == i2)).astype(jnp.int32)
    c = cnt
    sh = 1
    while sh < T:
        c = c + jnp.concatenate(
            [jnp.zeros((sh, E), jnp.int32), c[:T - sh]], axis=0)
        sh *= 2
    c_ex = c - cnt
    counts = c[T - 1:T, :]
    nblk = (counts + BM - 1) // BM
    cb = nblk
    for shl in (1, 2, 4):
        cb = cb + jnp.concatenate(
            [jnp.zeros((1, shl), jnp.int32), cb[:, :E - shl]], axis=1)
    pad_off = (cb - nblk) * BM
    posmat = pad_off + c_ex
    pos0 = jnp.sum(jnp.where(iota8 == i1, posmat, 0), axis=1, keepdims=True)
    pos1 = jnp.sum(jnp.where(iota8 == i2, posmat, 0), axis=1, keepdims=True)
    pos2_ref[...] = jnp.concatenate([pos0, pos1], axis=1)

    bi32 = lax.broadcasted_iota(jnp.int32, (MR, E), 0)
    eb = jnp.clip(jnp.sum((bi32 >= cb).astype(jnp.int32), axis=1,
                          keepdims=True), 0, E - 1)
    ri = lax.broadcasted_iota(jnp.int32, (MR, 1), 0)
    total = cb[:, E - 1:E]
    meta_col = jnp.where(ri == NB, total, eb)
    change = (eb != jnp.concatenate([eb[:1], eb[:MR - 1]], axis=0)).astype(jnp.int32)
    oc = change
    for shr in (1, 2, 4, 8, 16, 32):
        oc = oc + jnp.concatenate(
            [jnp.zeros((shr, 1), jnp.int32), oc[:MR - shr]], axis=0)
    meta_ref[...] = jnp.concatenate(
        [meta_col, oc] + [meta_col] * (E - 2), axis=1)


@jax.jit
def _routing(router_logits):
    return pl.pallas_call(
        _routing_body,
        out_shape=[
            jax.ShapeDtypeStruct((T, K), jnp.int32),
            jax.ShapeDtypeStruct((T, K), jnp.float32),
            jax.ShapeDtypeStruct((MR, E), jnp.int32),
        ],
    )(router_logits)


def _gmm_body(meta_ref, x_ref, w13_ref, w2_ref, o_ref,
              w13_buf, w2_buf, s13a, s13b, s2a, s2b):
    b = pl.program_id(0)
    s13 = (s13a, s13b)
    s2 = (s2a, s2b)

    def fetch(e_idx, par_idx, wait):
        for p in range(2):
            @pl.when(par_idx == p)
            def _():
                c13 = pltpu.make_async_copy(
                    w13_ref.at[e_idx], w13_buf.at[p], s13[p])
                c2 = pltpu.make_async_copy(
                    w2_ref.at[e_idx], w2_buf.at[p], s2[p])
                if wait == 'start':
                    c13.start()
                    c2.start()
                else:
                    c13.wait()
                    c2.wait()

    cur = meta_ref[b, 0]
    parity = lax.rem(meta_ref[b, 1], 2)
    prev = meta_ref[jnp.maximum(b - 1, 0), 0]
    first = jnp.logical_or(b == 0, prev != cur)
    valid = b < meta_ref[NB, 0]

    @pl.when(jnp.logical_and(first, valid))
    def _():
        @pl.when(b == 0)
        def _():
            fetch(cur, parity, 'start')

        def cond(j):
            return jnp.logical_and(j < NB, meta_ref[j, 0] == cur)

        j = lax.while_loop(cond, lambda j: j + 1, b + 1)
        nxt = meta_ref[jnp.minimum(j, NB - 1), 0]
        fetch(cur, parity, 'wait')

        @pl.when(jnp.logical_and(nxt != cur, j < meta_ref[NB, 0]))
        def _():
            fetch(nxt, 1 - parity, 'start')

    @pl.when(valid)
    def _():
        xb = x_ref[...]
        for p in range(2):
            @pl.when(parity == p)
            def _():
                g = jax.lax.dot_general(
                    xb, w13_buf[p, :I, :], (((1,), (1,)), ((), ())),
                    preferred_element_type=jnp.float32)
                u = jax.lax.dot_general(
                    xb, w13_buf[p, I:, :], (((1,), (1,)), ((), ())),
                    preferred_element_type=jnp.float32)
                act = g * jax.lax.logistic(g) * u
                o_ref[...] = jax.lax.dot_general(
                    act, w2_buf[p], (((1,), (1,)), ((), ())),
                    preferred_element_type=jnp.float32)


@jax.jit
def _gmm(meta, x_sorted, w13, w2):
    grid_spec = pltpu.PrefetchScalarGridSpec(
        num_scalar_prefetch=1,
        grid=(NB,),
        in_specs=[
            pl.BlockSpec((BM, H), lambda b, m: (b, 0)),
            pl.BlockSpec(memory_space=pl.ANY),
            pl.BlockSpec(memory_space=pl.ANY),
        ],
        out_specs=pl.BlockSpec((BM, H), lambda b, m: (b, 0)),
        scratch_shapes=[
            pltpu.VMEM((2, 2 * I, H), jnp.float32),
            pltpu.VMEM((2, H, I), jnp.float32),
            pltpu.SemaphoreType.DMA,
            pltpu.SemaphoreType.DMA,
            pltpu.SemaphoreType.DMA,
            pltpu.SemaphoreType.DMA,
        ],
    )
    return pl.pallas_call(
        _gmm_body,
        grid_spec=grid_spec,
        out_shape=jax.ShapeDtypeStruct((NPAD, H), jnp.float32),
    )(meta, x_sorted, w13, w2)


def kernel(x, router_logits, w13, w2):
    pos2, w2o, meta_out = _routing(router_logits.astype(jnp.float32))
    pos3d = pos2.reshape(NW, NCH, CH)
    w3d = w2o.reshape(NW, NCH, CH)
    tok3d = (jnp.arange(TK, dtype=jnp.int32) // K).reshape(NW, NCH, CH)
    x_sorted = _make_dispatch()(x, pos3d, tok3d)
    y = _gmm(meta_out, x_sorted, w13, w2)
    out = _make_combine()(y, pos3d, w3d)
    return out


# speedup vs baseline: 1.4385x; 1.4385x over previous
import functools
import jax
import jax.numpy as jnp
from jax import lax
from jax.experimental import pallas as pl
from jax.experimental.pallas import tpu as pltpu
from jax.experimental.pallas import tpu_sc as plsc

E = 8
K = 2
H = 1024
I = 2048
T = 2048
TK = T * K

BM = 256
BN = 512
NB = 23
NPAD = NB * BM
NI = I // BN


NC = 2
NS = 16
NW = NC * NS
PW = TK // NW
CH = 32
NCH = PW // CH

@functools.lru_cache(maxsize=None)
def _make_dispatch():
    mesh = plsc.VectorSubcoreMesh(core_axis_name="c", subcore_axis_name="s")
    return functools.partial(
        pl.kernel,
        mesh=mesh,
        out_type=jax.ShapeDtypeStruct((NPAD, H), jnp.float32),
        scratch_types=[
            pltpu.VMEM((NCH, CH), jnp.int32),
            pltpu.VMEM((NCH, CH), jnp.int32),
            pltpu.VMEM((CH, H), jnp.float32),
            pltpu.VMEM((CH, H), jnp.float32),
            pltpu.SemaphoreType.DMA,
            pltpu.SemaphoreType.DMA,
            pltpu.SemaphoreType.DMA,
            pltpu.SemaphoreType.DMA,
        ],
    )(_dispatch_body)


def _dispatch_body(x_hbm, pos_hbm, tok_hbm, xs_hbm, pos_v, tok_v,
                   rows0, rows1, sg0, sg1, ss0, ss1):
    wid = lax.axis_index("s") * NC + lax.axis_index("c")
    bufs = (rows0, rows1)
    sgs = (sg0, sg1)
    sss = (ss0, ss1)
    pltpu.sync_copy(pos_hbm.at[wid], pos_v)
    pltpu.sync_copy(tok_hbm.at[wid], tok_v)
    gath = {0: pltpu.async_copy(x_hbm.at[tok_v.at[0]], bufs[0], sgs[0])}
    scat = {}
    for j in range(NCH):
        gath[j].wait()
        if j + 1 < NCH:
            if j - 1 >= 0:
                scat[j - 1].wait()
            gath[j + 1] = pltpu.async_copy(
                x_hbm.at[tok_v.at[j + 1]], bufs[(j + 1) % 2], sgs[(j + 1) % 2])
        scat[j] = pltpu.async_copy(
            bufs[j % 2], xs_hbm.at[pos_v.at[j]], sss[j % 2])
    scat[NCH - 2].wait()
    scat[NCH - 1].wait()


CT = CH // K
HV = H // 16


@functools.lru_cache(maxsize=None)
def _make_combine():
    mesh = plsc.VectorSubcoreMesh(core_axis_name="c", subcore_axis_name="s")
    return functools.partial(
        pl.kernel,
        mesh=mesh,
        out_type=jax.ShapeDtypeStruct((T, H), jnp.float32),
        scratch_types=[
            pltpu.VMEM((NCH, CH), jnp.int32),
            pltpu.VMEM((NCH, CH), jnp.float32),
            pltpu.VMEM((CH, H), jnp.float32),
            pltpu.VMEM((CH, H), jnp.float32),
            pltpu.VMEM((CT, H), jnp.float32),
            pltpu.SemaphoreType.DMA,
            pltpu.SemaphoreType.DMA,
        ],
    )(_combine_body)


def _combine_body(y_hbm, pos_hbm, w_hbm, out_hbm, pos_v, w_v,
                  rows0, rows1, out_v, sem0, sem1):
    wid = lax.axis_index("s") * NC + lax.axis_index("c")
    bufs = (rows0, rows1)
    sems = (sem0, sem1)
    pltpu.sync_copy(pos_hbm.at[wid], pos_v)
    pltpu.sync_copy(w_hbm.at[wid], w_v)
    gath = {0: pltpu.async_copy(y_hbm.at[pos_v.at[0]], bufs[0], sems[0])}
    for j in range(NCH):
        gath[j].wait()
        if j + 1 < NCH:
            gath[j + 1] = pltpu.async_copy(
                y_hbm.at[pos_v.at[j + 1]], bufs[(j + 1) % 2], sems[(j + 1) % 2])
        rows_v = bufs[j % 2]
        wrow_a = w_v[j, pl.ds(0, 16)]
        wrow_b = w_v[j, pl.ds(16, 16)]
        for tt in range(CT):
            wrow = wrow_a if tt < CT // 2 else wrow_b
            w0 = wrow[(2 * tt) % 16]
            w1 = wrow[(2 * tt + 1) % 16]

            def body(kk, _):
                a = rows_v[2 * tt, pl.ds(kk * 16, 16)]
                b = rows_v[2 * tt + 1, pl.ds(kk * 16, 16)]
                out_v[tt, pl.ds(kk * 16, 16)] = a * w0 + b * w1
                return 0

            lax.fori_loop(0, HV, body, 0)
        base = pl.multiple_of(wid * (PW // K) + j * CT, CT)
        pltpu.sync_copy(out_v, out_hbm.at[pl.ds(base, CT)])


def _routing_body(lg_ref, pos2_ref, w2o_ref, meta_ref):
    lg = lg_ref[...]
    m = jnp.max(lg, axis=1, keepdims=True)
    p = jnp.exp(lg - m)
    probs = p / jnp.sum(p, axis=1, keepdims=True)

    iota8 = lax.broadcasted_iota(jnp.int32, (T, E), 1)
    m1 = jnp.max(probs, axis=1, keepdims=True)
    i1 = jnp.min(jnp.where(probs == m1, iota8, E), axis=1, keepdims=True)
    probs2 = jnp.where(iota8 == i1, -1.0, probs)
    m2 = jnp.max(probs2, axis=1, keepdims=True)
    i2 = jnp.min(jnp.where(probs2 == m2, iota8, E), axis=1, keepdims=True)
    denom = m1 + m2
    w2o_ref[...] = jnp.concatenate([m1 / denom, m2 / denom], axis=1)

    cnt = ((iota8 == i1) | (iota8 == i2)).astype(jnp.int32)
    c = cnt
    sh = 1
    while sh < T:
        c = c + jnp.concatenate(
            [jnp.zeros((sh, E), jnp.int32), c[:T - sh]], axis=0)
        sh *= 2
    c_ex = c - cnt
    counts = c[T - 1:T, :]
    nblk = (counts + BM - 1) // BM
    cb = nblk
    for shl in (1, 2, 4):
        cb = cb + jnp.concatenate(
            [jnp.zeros((1, shl), jnp.int32), cb[:, :E - shl]], axis=1)
    pad_off = (cb - nblk) * BM
    posmat = pad_off + c_ex
    pos0 = jnp.sum(jnp.where(iota8 == i1, posmat, 0), axis=1, keepdims=True)
    pos1 = jnp.sum(jnp.where(iota8 == i2, posmat, 0), axis=1, keepdims=True)
    pos2_ref[...] = jnp.concatenate([pos0, pos1], axis=1)

    bi32 = lax.broadcasted_iota(jnp.int32, (32, E), 0)
    eb = jnp.clip(jnp.sum((bi32 >= cb).astype(jnp.int32), axis=1,
                          keepdims=True), 0, E - 1)
    ri = lax.broadcasted_iota(jnp.int32, (32, 1), 0)
    total = cb[:, E - 1:E]
    meta_col = jnp.where(ri == NB, total, eb)
    change = (eb != jnp.concatenate([eb[:1], eb[:31]], axis=0)).astype(jnp.int32)
    oc = change
    for shr in (1, 2, 4, 8, 16):
        oc = oc + jnp.concatenate(
            [jnp.zeros((shr, 1), jnp.int32), oc[:32 - shr]], axis=0)
    meta_ref[...] = jnp.concatenate(
        [meta_col, oc] + [meta_col] * (E - 2), axis=1)


@jax.jit
def _routing(router_logits):
    return pl.pallas_call(
        _routing_body,
        out_shape=[
            jax.ShapeDtypeStruct((T, K), jnp.int32),
            jax.ShapeDtypeStruct((T, K), jnp.float32),
            jax.ShapeDtypeStruct((32, E), jnp.int32),
        ],
    )(router_logits)


def _gmm_body(meta_ref, x_ref, w13_ref, w2_ref, o_ref,
              w13_buf, w2_buf, s13a, s13b, s2a, s2b):
    b = pl.program_id(0)
    s13 = (s13a, s13b)
    s2 = (s2a, s2b)

    def fetch(e_idx, par_idx, wait):
        for p in range(2):
            @pl.when(par_idx == p)
            def _():
                c13 = pltpu.make_async_copy(
                    w13_ref.at[e_idx], w13_buf.at[p], s13[p])
                c2 = pltpu.make_async_copy(
                    w2_ref.at[e_idx], w2_buf.at[p], s2[p])
                if wait == 'start':
                    c13.start()
                    c2.start()
                else:
                    c13.wait()
                    c2.wait()

    cur = meta_ref[b, 0]
    parity = lax.rem(meta_ref[b, 1], 2)
    prev = meta_ref[jnp.maximum(b - 1, 0), 0]
    first = jnp.logical_or(b == 0, prev != cur)
    valid = b < meta_ref[NB, 0]

    @pl.when(jnp.logical_and(first, valid))
    def _():
        @pl.when(b == 0)
        def _():
            fetch(cur, parity, 'start')

        def cond(j):
            return jnp.logical_and(j < NB, meta_ref[j, 0] == cur)

        j = lax.while_loop(cond, lambda j: j + 1, b + 1)
        nxt = meta_ref[jnp.minimum(j, NB - 1), 0]
        fetch(cur, parity, 'wait')

        @pl.when(jnp.logical_and(nxt != cur, j < meta_ref[NB, 0]))
        def _():
            fetch(nxt, 1 - parity, 'start')

    @pl.when(valid)
    def _():
        xb = x_ref[...]
        for p in range(2):
            @pl.when(parity == p)
            def _():
                g = jax.lax.dot_general(
                    xb, w13_buf[p, :I, :], (((1,), (1,)), ((), ())),
                    preferred_element_type=jnp.float32)
                u = jax.lax.dot_general(
                    xb, w13_buf[p, I:, :], (((1,), (1,)), ((), ())),
                    preferred_element_type=jnp.float32)
                act = g * jax.lax.logistic(g) * u
                o_ref[...] = jax.lax.dot_general(
                    act, w2_buf[p], (((1,), (1,)), ((), ())),
                    preferred_element_type=jnp.float32)


@jax.jit
def _gmm(meta, x_sorted, w13, w2):
    grid_spec = pltpu.PrefetchScalarGridSpec(
        num_scalar_prefetch=1,
        grid=(NB,),
        in_specs=[
            pl.BlockSpec((BM, H), lambda b, m: (b, 0)),
            pl.BlockSpec(memory_space=pl.ANY),
            pl.BlockSpec(memory_space=pl.ANY),
        ],
        out_specs=pl.BlockSpec((BM, H), lambda b, m: (b, 0)),
        scratch_shapes=[
            pltpu.VMEM((2, 2 * I, H), jnp.float32),
            pltpu.VMEM((2, H, I), jnp.float32),
            pltpu.SemaphoreType.DMA,
            pltpu.SemaphoreType.DMA,
            pltpu.SemaphoreType.DMA,
            pltpu.SemaphoreType.DMA,
        ],
    )
    return pl.pallas_call(
        _gmm_body,
        grid_spec=grid_spec,
        out_shape=jax.ShapeDtypeStruct((NPAD, H), jnp.float32),
    )(meta, x_sorted, w13, w2)


def kernel(x, router_logits, w13, w2):
    pos2, w2o, meta_out = _routing(router_logits.astype(jnp.float32))
    pos3d = pos2.reshape(NW, NCH, CH)
    w3d = w2o.reshape(NW, NCH, CH)
    tok3d = (jnp.arange(TK, dtype=jnp.int32) // K).reshape(NW, NCH, CH)
    x_sorted = _make_dispatch()(x, pos3d, tok3d)
    y = _gmm(meta_out, x_sorted, w13, w2)
    out = _make_combine()(y, pos3d, w3d)
    return out
